# baseline (speedup 1.0000x reference)
import functools

import jax
import jax.numpy as jnp
from jax import lax
from jax.experimental import pallas as pl
from jax.experimental.pallas import tpu as pltpu
from jax.experimental.pallas import tpu_sc as plsc

N = 10000
E = 160000
D = 256
DH = 128
G = 64

NB = 400
EB = 2000
NSUB = 16
EPT = E // NSUB
CE = 40
NCH = EPT // CE
NPAD = 10240
RPT = NPAD // NSUB

_HI = jax.lax.Precision.HIGHEST


def _dot(a, b):
    return jnp.dot(a, b, preferred_element_type=jnp.float32)


def _pre_body(x_ref, ea_ref, wm_ref, pm_ref, we_ref, pe_ref,
              hr_ref, hf_ref, er_ref, ef_ref):
    i = pl.program_id(0)

    @pl.when(i < N // NB)
    def _():
        xb = x_ref[...]
        hr = _dot(xb, wm_ref[...])
        hr_ref[0] = hr[:, :DH]
        hr_ref[1] = hr[:, DH:]
        hf = _dot(xb, pm_ref[...])
        hf_ref[0] = hf[:, :DH]
        hf_ref[1] = hf[:, DH:]

    eb = ea_ref[...]
    er = _dot(eb, we_ref[...])
    er_ref[0] = er[:, :DH]
    er_ref[1] = er[:, DH:]
    ef = _dot(eb, pe_ref[...])
    ef_ref[0] = ef[:, :DH]
    ef_ref[1] = ef[:, DH:]


def _pre(x, edge_attr, Wm, Pm, We, Pe):
    de = edge_attr.shape[1]
    nh = N // NB
    full = lambda r, c: pl.BlockSpec((r, c), lambda i: (0, 0))
    hmap = lambda i: (jnp.minimum(i, nh - 1), 0)
    return pl.pallas_call(
        _pre_body,
        grid=(E // EB,),
        in_specs=[
            pl.BlockSpec((NB, D), hmap),
            pl.BlockSpec((EB, de), lambda i: (i, 0)),
            full(D, D), full(D, D), full(de, D), full(de, D),
        ],
        out_specs=[pl.BlockSpec((2, NB, DH), lambda i: (0, *hmap(i)))] * 2
        + [pl.BlockSpec((2, EB, DH), lambda i: (0, i, 0))] * 2,
        out_shape=[jax.ShapeDtypeStruct((2, N, DH), jnp.float32)] * 2
        + [jax.ShapeDtypeStruct((2, E, DH), jnp.float32)] * 2,
    )(x, edge_attr, Wm, Pm, We, Pe)


STRIP = 2000
LCAP = 11264
CO = 32
NQUAD = LCAP // CO // 4


def _sc_body(src_hbm, dst_hbm,
             h_rep, h_ft, ew_rep, ew_ft,
             o_rep, o_ft,
             gatb, ewb, eidl, dstl, dbuf, sbuf, s0, s1, s2, s3,
             acc,
             src_sems, gat_sems, ew_sems):
    srcv = [s0, s1, s2, s3]
    cid = lax.axis_index("c")
    sid = lax.axis_index("s")
    lo = sid * RPT

    def strip(sno, cnt):
        pltpu.sync_copy(dst_hbm.at[pl.ds(sno * STRIP, STRIP)], dbuf)

        def vec(v, cnt):
            lane = lax.iota(jnp.int32, 16)
            d16 = dbuf[pl.ds(v * 16, 16)]
            eid16 = lane + (sno * STRIP + v * 16)
            m = jnp.logical_and(d16 >= lo, d16 < lo + RPT)
            mi = m.astype(jnp.int32)
            cs = plsc.cumsum(mi)
            pos = jnp.where(m, cnt + cs - mi, LCAP + lane)
            plsc.store_scatter(eidl, [pos], eid16)
            plsc.store_scatter(dstl, [pos], d16 - lo)
            return cnt + cs[15]

        return lax.fori_loop(0, STRIP // 16, vec, cnt)

    cnt = lax.fori_loop(0, E // STRIP, strip, 0)
    eidl[pl.ds(cnt, 16)] = jnp.zeros((16,), jnp.int32)
    eidl[pl.ds(cnt + 16, 16)] = jnp.zeros((16,), jnp.int32)
    dstl[pl.ds(cnt, 16)] = jnp.full((16,), RPT, jnp.int32)
    dstl[pl.ds(cnt + 16, 16)] = jnp.full((16,), RPT, jnp.int32)
    nch = (cnt + CO - 1) // CO

    def process(h_hbm, ew_hbm, out_hbm):
        def start_src(c, b4):
            pltpu.async_copy(src_hbm.at[eidl.at[pl.ds(c * CO, CO)]],
                             srcv[b4], src_sems.at[b4])

        def wait_src(c, b4):
            pltpu.make_async_copy(src_hbm.at[eidl.at[pl.ds(c * CO, CO)]],
                                  srcv[b4], src_sems.at[b4]).wait()

        def start_gat(c, b4, b):
            pltpu.async_copy(h_hbm.at[cid].at[srcv[b4]], gatb.at[b],
                             gat_sems.at[b])
            pltpu.async_copy(ew_hbm.at[cid].at[eidl.at[pl.ds(c * CO, CO)]],
                             ewb.at[b], ew_sems.at[b])

        def wait_gat(c, b4, b):
            pltpu.make_async_copy(h_hbm.at[cid].at[srcv[b4]], gatb.at[b],
                                  gat_sems.at[b]).wait()
            pltpu.make_async_copy(ew_hbm.at[cid].at[eidl.at[pl.ds(c * CO, CO)]],
                                  ewb.at[b], ew_sems.at[b]).wait()

        def zrow(i, _):
            for j in range(DH // 16):
                acc[i, pl.ds(j * 16, 16)] = jnp.zeros((16,), jnp.float32)
            return 0

        lax.fori_loop(0, RPT + 8, zrow, 0)

        for c in range(4):
            @pl.when(c < nch)
            def _():
                start_src(c, c)
        for c in range(2):
            @pl.when(c < nch)
            def _():
                wait_src(c, c)
                start_gat(c, c, c % 2)

        def quad(g, _):
            for b in range(4):
                c = g * 4 + b

                @pl.when(c < nch)
                def _():
                    wait_gat(c, b, b % 2)

                    def rgrp(gi, _):
                        dvec = dstl[pl.ds(c * CO + gi * 16, 16)]
                        for ln in range(16):
                            i = gi * 16 + ln
                            d = dvec[ln]
                            for j in range(DH // 16):
                                s = pl.ds(j * 16, 16)
                                acc[d, s] = acc[d, s] + jnp.maximum(
                                    gatb[b % 2, i, s] + ewb[b % 2, i, s], 0.0)
                        return 0

                    lax.fori_loop(0, CO // 16, rgrp, 0)

                    @pl.when(c + 2 < nch)
                    def _():
                        wait_src(c + 2, (b + 2) % 4)
                        start_gat(c + 2, (b + 2) % 4, b % 2)

                    @pl.when(c + 4 < nch)
                    def _():
                        start_src(c + 4, b)
            return 0

        lax.fori_loop(0, NQUAD, quad, 0)
        pltpu.sync_copy(acc.at[pl.ds(0, RPT)],
                        out_hbm.at[cid].at[pl.ds(lo, RPT)])

    process(h_rep, ew_rep, o_rep)
    process(h_ft, ew_ft, o_ft)


def _sc_edge(edge_index, h4, ew4):
    f = pl.kernel(
        _sc_body,
        out_type=[jax.ShapeDtypeStruct((2, NPAD, DH), jnp.float32)] * 2,
        mesh=plsc.VectorSubcoreMesh(core_axis_name="c", subcore_axis_name="s",
                                    num_cores=2, num_subcores=NSUB),
        compiler_params=pltpu.CompilerParams(needs_layout_passes=False),
        scratch_types=[
            pltpu.VMEM((2, CO, DH), jnp.float32),
            pltpu.VMEM((2, CO, DH), jnp.float32),
            pltpu.VMEM((LCAP + 16,), jnp.int32),
            pltpu.VMEM((LCAP + 16,), jnp.int32),
            pltpu.VMEM((STRIP,), jnp.int32),
            pltpu.VMEM((32,), jnp.int32),
        ] + [pltpu.VMEM((CO,), jnp.int32)] * 4 + [
            pltpu.VMEM((RPT + 8, DH), jnp.float32),
            pltpu.SemaphoreType.DMA((4,)),
            pltpu.SemaphoreType.DMA((2,)),
            pltpu.SemaphoreType.DMA((2,)),
        ],
    )
    return f(edge_index[0], edge_index[1], *h4, *ew4)


def _post_body(x_ref, arl_ref, arh_ref, afl_ref, afh_ref, b_ref,
               ws_ref, wu_ref, ps_ref, pu_ref,
               w1_ref, b1_ref, w2_ref, b2_ref, w3_ref, b3_ref,
               feats_ref, gr_ref, sc_ref, sums, cnts):
    i = pl.program_id(0)

    @pl.when(i == 0)
    def _():
        sums[...] = jnp.zeros_like(sums)
        cnts[...] = jnp.zeros_like(cnts)

    xb = x_ref[...]
    wu = wu_ref[...]
    rep = jnp.maximum(
        _dot(arl_ref[0], wu[:DH, :]) + _dot(arh_ref[0], wu[DH:, :])
        + _dot(xb, ws_ref[...]), 0.0)
    pu = pu_ref[...]
    feats_ref[...] = jnp.maximum(
        _dot(afl_ref[0], pu[:DH, :]) + _dot(afh_ref[0], pu[DH:, :])
        + _dot(xb, ps_ref[...]), 0.0)

    bvals = b_ref[0, 0, :]
    onehot = (bvals[:, None]
              == jax.lax.broadcasted_iota(jnp.int32, (NB, G), 1)
              ).astype(jnp.float32)
    sums[...] += jax.lax.dot_general(
        onehot, rep, dimension_numbers=(((0,), (0,)), ((), ())),
        preferred_element_type=jnp.float32, precision=_HI)
    cnts[...] += jnp.broadcast_to(jnp.sum(onehot, axis=0)[:, None], (G, D))

    @pl.when(i == pl.num_programs(0) - 1)
    def _():
        gr = sums[...] / jnp.maximum(cnts[...], 1.0)
        gr_ref[...] = gr
        h1 = jnp.maximum(_dot(gr, w1_ref[...]) + b1_ref[...], 0.0)
        h2 = jnp.maximum(_dot(h1, w2_ref[...]) + b2_ref[...], 0.0)
        sc_ref[...] = _dot(h2, w3_ref[...])[:, :1] + b3_ref[...]


def _post(x, aggs, batch3, Ws, Wu, Ps, Pu, W1, b1, W2, b2, w3row, b3):
    n_blocks = N // NB
    full = lambda r, c: pl.BlockSpec((r, c), lambda i: (0, 0))
    return pl.pallas_call(
        _post_body,
        grid=(n_blocks,),
        in_specs=[
            pl.BlockSpec((NB, D), lambda i: (i, 0)),
            pl.BlockSpec((1, NB, DH), lambda i: (0, i, 0)),
            pl.BlockSpec((1, NB, DH), lambda i: (1, i, 0)),
            pl.BlockSpec((1, NB, DH), lambda i: (0, i, 0)),
            pl.BlockSpec((1, NB, DH), lambda i: (1, i, 0)),
            pl.BlockSpec((1, 1, NB), lambda i: (i, 0, 0)),
            full(D, D), full(D, D), full(D, D), full(D, D),
            full(D, D), full(1, D), full(D, D), full(1, D),
            full(D, DH), full(1, 1),
        ],
        out_specs=[
            pl.BlockSpec((NB, D), lambda i: (i, 0)),
            pl.BlockSpec((G, D), lambda i: (0, 0)),
            pl.BlockSpec((G, 1), lambda i: (0, 0)),
        ],
        out_shape=[
            jax.ShapeDtypeStruct((N, D), jnp.float32),
            jax.ShapeDtypeStruct((G, D), jnp.float32),
            jax.ShapeDtypeStruct((G, 1), jnp.float32),
        ],
        scratch_shapes=[
            pltpu.VMEM((G, D), jnp.float32),
            pltpu.VMEM((G, D), jnp.float32),
        ],
    )(x, aggs[0], aggs[0], aggs[1], aggs[1], batch3,
      Ws, Wu, Ps, Pu, W1, b1, W2, b2, w3row, b3)


def kernel(x, edge_index, edge_attr, batch,
           Wm, We, Ws, Wu, Pm, Pe, Ps, Pu, W1, b1, W2, b2, W3, b3):
    hr_st, hf_st, er_st, ef_st = _pre(x, edge_attr, Wm, Pm, We, Pe)
    aggs = _sc_edge(edge_index, (hr_st, hf_st), (er_st, ef_st))
    batch3 = batch.reshape(N // NB, 1, NB)
    w3pad = jnp.pad(W3, ((0, 0), (0, DH - W3.shape[1])))
    feats, gr, sc = _post(
        x, aggs, batch3, Ws, Wu, Ps, Pu,
        W1, b1.reshape(1, D), W2, b2.reshape(1, D),
        w3pad, b3.reshape(1, 1))
    return (sc[:, 0], gr, feats)


# speedup vs baseline: 2.2163x; 2.2163x over previous
import functools

import jax
import jax.numpy as jnp
from jax import lax
from jax.experimental import pallas as pl
from jax.experimental.pallas import tpu as pltpu
from jax.experimental.pallas import tpu_sc as plsc

N = 10000
E = 160000
D = 256
DH = 128
G = 64

NB = 400
EB = 2000
NSUB = 16
EPT = E // NSUB
CE = 40
NCH = EPT // CE
NPAD = 10240
RPT = NPAD // NSUB

_HI = jax.lax.Precision.HIGHEST


def _dot(a, b):
    return jnp.dot(a, b, preferred_element_type=jnp.float32)


def _pre_body(x_ref, ea_ref, wm_ref, pm_ref, we_ref, pe_ref,
              hr_ref, hf_ref, er_ref, ef_ref):
    i = pl.program_id(0)

    @pl.when(i < N // NB)
    def _():
        xb = x_ref[...]
        hr = _dot(xb, wm_ref[...])
        hr_ref[0] = hr[:, :DH]
        hr_ref[1] = hr[:, DH:]
        hf = _dot(xb, pm_ref[...])
        hf_ref[0] = hf[:, :DH]
        hf_ref[1] = hf[:, DH:]

    eb = ea_ref[...]
    er = _dot(eb, we_ref[...])
    er_ref[0] = er[:, :DH]
    er_ref[1] = er[:, DH:]
    ef = _dot(eb, pe_ref[...])
    ef_ref[0] = ef[:, :DH]
    ef_ref[1] = ef[:, DH:]


def _pre(x, edge_attr, Wm, Pm, We, Pe):
    de = edge_attr.shape[1]
    nh = N // NB
    full = lambda r, c: pl.BlockSpec((r, c), lambda i: (0, 0))
    hmap = lambda i: (jnp.minimum(i, nh - 1), 0)
    return pl.pallas_call(
        _pre_body,
        grid=(E // EB,),
        in_specs=[
            pl.BlockSpec((NB, D), hmap),
            pl.BlockSpec((EB, de), lambda i: (i, 0)),
            full(D, D), full(D, D), full(de, D), full(de, D),
        ],
        out_specs=[pl.BlockSpec((2, NB, DH), lambda i: (0, *hmap(i)))] * 2
        + [pl.BlockSpec((2, EB, DH), lambda i: (0, i, 0))] * 2,
        out_shape=[jax.ShapeDtypeStruct((2, N, DH), jnp.float32)] * 2
        + [jax.ShapeDtypeStruct((2, E, DH), jnp.float32)] * 2,
    )(x, edge_attr, Wm, Pm, We, Pe)


STRIP = 1600
LCAP = 10944
CO = 24
NQUAD = LCAP // CO // 4


def _sc_body(src_hbm, dst_hbm,
             h_rep, h_ft, ew_rep, ew_ft,
             o_rep, o_ft,
             gatb, ewb, eidl, dstl, dbuf, sbuf, s0, s1, s2, s3,
             zbuf, acc,
             src_sems, gat_sems, ew_sems, sc_sem):
    srcv = [s0, s1, s2, s3]
    cid = lax.axis_index("c")
    sid = lax.axis_index("s")
    lo = sid * RPT

    def strip(sno, cnt):
        pltpu.sync_copy(dst_hbm.at[pl.ds(sno * STRIP, STRIP)], dbuf)

        def vec(v, cnt):
            lane = lax.iota(jnp.int32, 16)
            d16 = dbuf[pl.ds(v * 16, 16)]
            eid16 = lane + (sno * STRIP + v * 16)
            m = jnp.logical_and(d16 >= lo, d16 < lo + RPT)
            mi = m.astype(jnp.int32)
            cs = plsc.cumsum(mi)
            pos = jnp.where(m, cnt + cs - mi, LCAP + lane)
            plsc.store_scatter(eidl, [pos], eid16)
            plsc.store_scatter(dstl, [pos], d16)
            return cnt + cs[15]

        return lax.fori_loop(0, STRIP // 16, vec, cnt)

    cnt = lax.fori_loop(0, E // STRIP, strip, 0)
    eidl[pl.ds(cnt, 16)] = jnp.zeros((16,), jnp.int32)
    eidl[pl.ds(cnt + 16, 16)] = jnp.zeros((16,), jnp.int32)
    dstl[pl.ds(cnt, 16)] = jnp.full((16,), N, jnp.int32)
    dstl[pl.ds(cnt + 16, 16)] = jnp.full((16,), N, jnp.int32)
    nch = (cnt + CO - 1) // CO

    def process(h_hbm, ew_hbm, out_hbm):
        def start_src(c, b4):
            pltpu.async_copy(src_hbm.at[eidl.at[pl.ds(c * CO, CO)]],
                             srcv[b4], src_sems.at[b4])

        def wait_src(c, b4):
            pltpu.make_async_copy(src_hbm.at[eidl.at[pl.ds(c * CO, CO)]],
                                  srcv[b4], src_sems.at[b4]).wait()

        def start_gat(c, b4, b):
            pltpu.async_copy(h_hbm.at[cid].at[srcv[b4]], gatb.at[b4],
                             gat_sems.at[b4])
            pltpu.async_copy(ew_hbm.at[cid].at[eidl.at[pl.ds(c * CO, CO)]],
                             ewb.at[b], ew_sems.at[b])


        def wait_gat(c, b4, b):
            pltpu.make_async_copy(h_hbm.at[cid].at[srcv[b4]], gatb.at[b4],
                                  gat_sems.at[b4]).wait()
            pltpu.make_async_copy(ew_hbm.at[cid].at[eidl.at[pl.ds(c * CO, CO)]],
                                  ewb.at[b], ew_sems.at[b]).wait()


        def start_sc(c, b4):
            pltpu.async_copy(gatb.at[b4], acc.at[dstl.at[pl.ds(c * CO, CO)]],
                             sc_sem, add=True)

        def wait_sc_one():
            pltpu.make_async_copy(gatb.at[0], acc.at[dstl.at[pl.ds(0, CO)]],
                                  sc_sem).wait()

        def zrow(i, _):
            for j in range(DH // 16):
                zbuf[i, pl.ds(j * 16, 16)] = jnp.zeros((16,), jnp.float32)
            return 0

        lax.fori_loop(0, zbuf.shape[0], zrow, 0)
        for k in range(RPT // zbuf.shape[0]):
            pltpu.sync_copy(zbuf, acc.at[pl.ds(lo + k * zbuf.shape[0],
                                               zbuf.shape[0])])

        for c in range(4):
            @pl.when(c < nch)
            def _():
                start_src(c, c)
        for c in range(2):
            @pl.when(c < nch)
            def _():
                wait_src(c, c)
                start_gat(c, c, c % 2)

        def quad(g, _):
            for b in range(4):
                c = g * 4 + b

                @pl.when(c < nch)
                def _():
                    wait_gat(c, b, b % 2)

                    def row(i, _):
                        for j in range(DH // 16):
                            s = pl.ds(j * 16, 16)
                            gatb[b, i, s] = jnp.maximum(
                                gatb[b, i, s] + ewb[b % 2, i, s], 0.0)
                        return 0

                    lax.fori_loop(0, CO, row, 0)

                    @pl.when(c > 0)
                    def _():
                        wait_sc_one()

                    start_sc(c, b)

                    @pl.when(c + 2 < nch)
                    def _():
                        wait_src(c + 2, (b + 2) % 4)
                        start_gat(c + 2, (b + 2) % 4, b % 2)

                    @pl.when(c + 4 < nch)
                    def _():
                        start_src(c + 4, b)
            return 0

        lax.fori_loop(0, NQUAD, quad, 0)

        @pl.when(nch > 0)
        def _():
            wait_sc_one()

        pltpu.sync_copy(acc.at[pl.ds(lo, RPT)],
                        out_hbm.at[cid].at[pl.ds(lo, RPT)])

    process(h_rep, ew_rep, o_rep)
    process(h_ft, ew_ft, o_ft)


def _sc_edge(edge_index, h4, ew4):
    f = pl.kernel(
        _sc_body,
        out_type=[jax.ShapeDtypeStruct((2, NPAD, DH), jnp.float32)] * 2,
        mesh=plsc.VectorSubcoreMesh(core_axis_name="c", subcore_axis_name="s",
                                    num_cores=2, num_subcores=NSUB),
        compiler_params=pltpu.CompilerParams(needs_layout_passes=False),
        scratch_types=[
            pltpu.VMEM((4, CO, DH), jnp.float32),
            pltpu.VMEM((2, CO, DH), jnp.float32),
            pltpu.VMEM((LCAP + 16,), jnp.int32),
            pltpu.VMEM((LCAP + 16,), jnp.int32),
            pltpu.VMEM((STRIP,), jnp.int32),
            pltpu.VMEM((32,), jnp.int32),
        ] + [pltpu.VMEM((CO,), jnp.int32)] * 4 + [
            pltpu.VMEM((32, DH), jnp.float32),
            pltpu.VMEM_SHARED((NPAD, DH), jnp.float32),
            pltpu.SemaphoreType.DMA((4,)),
            pltpu.SemaphoreType.DMA((4,)),
            pltpu.SemaphoreType.DMA((2,)),
            pltpu.SemaphoreType.DMA,
        ],
    )
    return f(edge_index[0], edge_index[1], *h4, *ew4)


def _post_body(x_ref, arl_ref, arh_ref, afl_ref, afh_ref, b_ref,
               ws_ref, wu_ref, ps_ref, pu_ref,
               w1_ref, b1_ref, w2_ref, b2_ref, w3_ref, b3_ref,
               feats_ref, gr_ref, sc_ref, sums, cnts):
    i = pl.program_id(0)

    @pl.when(i == 0)
    def _():
        sums[...] = jnp.zeros_like(sums)
        cnts[...] = jnp.zeros_like(cnts)

    xb = x_ref[...]
    wu = wu_ref[...]
    rep = jnp.maximum(
        _dot(arl_ref[0], wu[:DH, :]) + _dot(arh_ref[0], wu[DH:, :])
        + _dot(xb, ws_ref[...]), 0.0)
    pu = pu_ref[...]
    feats_ref[...] = jnp.maximum(
        _dot(afl_ref[0], pu[:DH, :]) + _dot(afh_ref[0], pu[DH:, :])
        + _dot(xb, ps_ref[...]), 0.0)

    bvals = b_ref[0, 0, :]
    onehot = (bvals[:, None]
              == jax.lax.broadcasted_iota(jnp.int32, (NB, G), 1)
              ).astype(jnp.float32)
    sums[...] += jax.lax.dot_general(
        onehot, rep, dimension_numbers=(((0,), (0,)), ((), ())),
        preferred_element_type=jnp.float32, precision=_HI)
    cnts[...] += jnp.broadcast_to(jnp.sum(onehot, axis=0)[:, None], (G, D))

    @pl.when(i == pl.num_programs(0) - 1)
    def _():
        gr = sums[...] / jnp.maximum(cnts[...], 1.0)
        gr_ref[...] = gr
        h1 = jnp.maximum(_dot(gr, w1_ref[...]) + b1_ref[...], 0.0)
        h2 = jnp.maximum(_dot(h1, w2_ref[...]) + b2_ref[...], 0.0)
        sc_ref[...] = _dot(h2, w3_ref[...])[:, :1] + b3_ref[...]


def _post(x, aggs, batch3, Ws, Wu, Ps, Pu, W1, b1, W2, b2, w3row, b3):
    n_blocks = N // NB
    full = lambda r, c: pl.BlockSpec((r, c), lambda i: (0, 0))
    return pl.pallas_call(
        _post_body,
        grid=(n_blocks,),
        in_specs=[
            pl.BlockSpec((NB, D), lambda i: (i, 0)),
            pl.BlockSpec((1, NB, DH), lambda i: (0, i, 0)),
            pl.BlockSpec((1, NB, DH), lambda i: (1, i, 0)),
            pl.BlockSpec((1, NB, DH), lambda i: (0, i, 0)),
            pl.BlockSpec((1, NB, DH), lambda i: (1, i, 0)),
            pl.BlockSpec((1, 1, NB), lambda i: (i, 0, 0)),
            full(D, D), full(D, D), full(D, D), full(D, D),
            full(D, D), full(1, D), full(D, D), full(1, D),
            full(D, DH), full(1, 1),
        ],
        out_specs=[
            pl.BlockSpec((NB, D), lambda i: (i, 0)),
            pl.BlockSpec((G, D), lambda i: (0, 0)),
            pl.BlockSpec((G, 1), lambda i: (0, 0)),
        ],
        out_shape=[
            jax.ShapeDtypeStruct((N, D), jnp.float32),
            jax.ShapeDtypeStruct((G, D), jnp.float32),
            jax.ShapeDtypeStruct((G, 1), jnp.float32),
        ],
        scratch_shapes=[
            pltpu.VMEM((G, D), jnp.float32),
            pltpu.VMEM((G, D), jnp.float32),
        ],
    )(x, aggs[0], aggs[0], aggs[1], aggs[1], batch3,
      Ws, Wu, Ps, Pu, W1, b1, W2, b2, w3row, b3)


def kernel(x, edge_index, edge_attr, batch,
           Wm, We, Ws, Wu, Pm, Pe, Ps, Pu, W1, b1, W2, b2, W3, b3):
    hr_st, hf_st, er_st, ef_st = _pre(x, edge_attr, Wm, Pm, We, Pe)
    aggs = _sc_edge(edge_index, (hr_st, hf_st), (er_st, ef_st))
    batch3 = batch.reshape(N // NB, 1, NB)
    w3pad = jnp.pad(W3, ((0, 0), (0, DH - W3.shape[1])))
    feats, gr, sc = _post(
        x, aggs, batch3, Ws, Wu, Ps, Pu,
        W1, b1.reshape(1, D), W2, b2.reshape(1, D),
        w3pad, b3.reshape(1, 1))
    return (sc[:, 0], gr, feats)


# speedup vs baseline: 2.3716x; 1.0701x over previous
import functools

import jax
import jax.numpy as jnp
from jax import lax
from jax.experimental import pallas as pl
from jax.experimental.pallas import tpu as pltpu
from jax.experimental.pallas import tpu_sc as plsc

N = 10000
E = 160000
D = 256
DH = 128
G = 64

NB = 400
EB = 2000
NSUB = 16
EPT = E // NSUB
CE = 40
NCH = EPT // CE
NPAD = 10240
RPT = NPAD // NSUB

_HI = jax.lax.Precision.HIGHEST


def _dot(a, b):
    return jnp.dot(a, b, preferred_element_type=jnp.float32)


def _pre_body(x_ref, ea_ref, wm_ref, pm_ref, we_ref, pe_ref,
              hr_ref, hf_ref, er_ref, ef_ref):
    i = pl.program_id(0)

    @pl.when(i < N // NB)
    def _():
        xb = x_ref[...]
        hr = _dot(xb, wm_ref[...])
        hr_ref[0] = hr[:, :DH]
        hr_ref[1] = hr[:, DH:]
        hf = _dot(xb, pm_ref[...])
        hf_ref[0] = hf[:, :DH]
        hf_ref[1] = hf[:, DH:]

    eb = ea_ref[...]
    er = _dot(eb, we_ref[...])
    er_ref[0] = er[:, :DH]
    er_ref[1] = er[:, DH:]
    ef = _dot(eb, pe_ref[...])
    ef_ref[0] = ef[:, :DH]
    ef_ref[1] = ef[:, DH:]


def _pre(x, edge_attr, Wm, Pm, We, Pe):
    de = edge_attr.shape[1]
    nh = N // NB
    full = lambda r, c: pl.BlockSpec((r, c), lambda i: (0, 0))
    hmap = lambda i: (jnp.minimum(i, nh - 1), 0)
    return pl.pallas_call(
        _pre_body,
        grid=(E // EB,),
        in_specs=[
            pl.BlockSpec((NB, D), hmap),
            pl.BlockSpec((EB, de), lambda i: (i, 0)),
            full(D, D), full(D, D), full(de, D), full(de, D),
        ],
        out_specs=[pl.BlockSpec((2, NB, DH), lambda i: (0, *hmap(i)))] * 2
        + [pl.BlockSpec((2, EB, DH), lambda i: (0, i, 0))] * 2,
        out_shape=[jax.ShapeDtypeStruct((2, N, DH), jnp.float32)] * 2
        + [jax.ShapeDtypeStruct((2, E, DH), jnp.float32)] * 2,
    )(x, edge_attr, Wm, Pm, We, Pe)


STRIP = 1600
LCAP = 10944
CO = 24
NQUAD = LCAP // CO // 4


def _sc_body(src_hbm, dst_hbm,
             h_rep, h_ft, ew_rep, ew_ft,
             o_rep, o_ft,
             gatb, ewb, eidl, dstl, dbuf0, dbuf1, sbuf, s0, s1, s2, s3,
             zbuf, acc,
             src_sems, gat_sems, ew_sems, sc_sem, db_sems):
    srcv = [s0, s1, s2, s3]
    cid = lax.axis_index("c")
    sid = lax.axis_index("s")
    lo = sid * RPT

    def dstrip(sno):
        return dst_hbm.at[pl.ds(sno * STRIP, STRIP)]

    dbv = [dbuf0, dbuf1]

    def strip2(g, cnt):
        for b in range(2):
            sno = g * 2 + b
            pltpu.make_async_copy(dstrip(sno), dbv[b], db_sems.at[b]).wait()

            @pl.when(sno + 1 < E // STRIP)
            def _():
                pltpu.async_copy(dstrip(sno + 1), dbv[1 - b],
                                 db_sems.at[1 - b])

            cnt = scan_strip(sno, b, cnt)
        return cnt

    def scan_strip(sno, b, cnt):
        def vec(v, cnt):
            lane = lax.iota(jnp.int32, 16)
            d16 = dbv[b][pl.ds(v * 16, 16)]
            eid16 = lane + (sno * STRIP + v * 16)
            m = jnp.logical_and(d16 >= lo, d16 < lo + RPT)
            mi = m.astype(jnp.int32)
            cs = plsc.cumsum(mi)
            pos = jnp.where(m, cnt + cs - mi, LCAP + lane)
            plsc.store_scatter(eidl, [pos], eid16)
            plsc.store_scatter(dstl, [pos], d16)
            return cnt + cs[15]

        return lax.fori_loop(0, STRIP // 16, vec, cnt)

    pltpu.async_copy(dstrip(0), dbv[0], db_sems.at[0])
    cnt = lax.fori_loop(0, E // STRIP // 2, strip2, 0)
    eidl[pl.ds(cnt, 16)] = jnp.zeros((16,), jnp.int32)
    eidl[pl.ds(cnt + 16, 16)] = jnp.zeros((16,), jnp.int32)
    dstl[pl.ds(cnt, 16)] = jnp.full((16,), N, jnp.int32)
    dstl[pl.ds(cnt + 16, 16)] = jnp.full((16,), N, jnp.int32)
    nch = (cnt + CO - 1) // CO

    def process(h_hbm, ew_hbm, out_hbm):
        def start_src(c, b4):
            pltpu.async_copy(src_hbm.at[eidl.at[pl.ds(c * CO, CO)]],
                             srcv[b4], src_sems.at[b4])

        def wait_src(c, b4):
            pltpu.make_async_copy(src_hbm.at[eidl.at[pl.ds(c * CO, CO)]],
                                  srcv[b4], src_sems.at[b4]).wait()

        def start_gat(c, b4, b):
            pltpu.async_copy(h_hbm.at[cid].at[srcv[b4]], gatb.at[b4],
                             gat_sems.at[b4])
            pltpu.async_copy(ew_hbm.at[cid].at[eidl.at[pl.ds(c * CO, CO)]],
                             ewb.at[b], ew_sems.at[b])


        def wait_gat(c, b4, b):
            pltpu.make_async_copy(h_hbm.at[cid].at[srcv[b4]], gatb.at[b4],
                                  gat_sems.at[b4]).wait()
            pltpu.make_async_copy(ew_hbm.at[cid].at[eidl.at[pl.ds(c * CO, CO)]],
                                  ewb.at[b], ew_sems.at[b]).wait()


        def start_sc(c, b4):
            pltpu.async_copy(gatb.at[b4], acc.at[dstl.at[pl.ds(c * CO, CO)]],
                             sc_sem, add=True)

        def wait_sc_one():
            pltpu.make_async_copy(gatb.at[0], acc.at[dstl.at[pl.ds(0, CO)]],
                                  sc_sem).wait()

        def zrow(i, _):
            for j in range(DH // 16):
                zbuf[i, pl.ds(j * 16, 16)] = jnp.zeros((16,), jnp.float32)
            return 0

        lax.fori_loop(0, zbuf.shape[0], zrow, 0)
        for k in range(RPT // zbuf.shape[0]):
            pltpu.sync_copy(zbuf, acc.at[pl.ds(lo + k * zbuf.shape[0],
                                               zbuf.shape[0])])

        for c in range(4):
            @pl.when(c < nch)
            def _():
                start_src(c, c)
        for c in range(2):
            @pl.when(c < nch)
            def _():
                wait_src(c, c)
                start_gat(c, c, c % 2)

        def quad(g, _):
            for b in range(4):
                c = g * 4 + b

                @pl.when(c < nch)
                def _():
                    wait_gat(c, b, b % 2)

                    def row(i, _):
                        for j in range(DH // 16):
                            s = pl.ds(j * 16, 16)
                            gatb[b, i, s] = jnp.maximum(
                                gatb[b, i, s] + ewb[b % 2, i, s], 0.0)
                        return 0

                    lax.fori_loop(0, CO, row, 0)

                    @pl.when(c > 0)
                    def _():
                        wait_sc_one()

                    start_sc(c, b)

                    @pl.when(c + 2 < nch)
                    def _():
                        wait_src(c + 2, (b + 2) % 4)
                        start_gat(c + 2, (b + 2) % 4, b % 2)

                    @pl.when(c + 4 < nch)
                    def _():
                        start_src(c + 4, b)
            return 0

        lax.fori_loop(0, NQUAD, quad, 0)

        @pl.when(nch > 0)
        def _():
            wait_sc_one()

        pltpu.sync_copy(acc.at[pl.ds(lo, RPT)],
                        out_hbm.at[cid].at[pl.ds(lo, RPT)])

    process(h_rep, ew_rep, o_rep)
    process(h_ft, ew_ft, o_ft)


def _sc_edge(edge_index, h4, ew4):
    f = pl.kernel(
        _sc_body,
        out_type=[jax.ShapeDtypeStruct((2, NPAD, DH), jnp.float32)] * 2,
        mesh=plsc.VectorSubcoreMesh(core_axis_name="c", subcore_axis_name="s",
                                    num_cores=2, num_subcores=NSUB),
        compiler_params=pltpu.CompilerParams(needs_layout_passes=False),
        scratch_types=[
            pltpu.VMEM((4, CO, DH), jnp.float32),
            pltpu.VMEM((2, CO, DH), jnp.float32),
            pltpu.VMEM((LCAP + 16,), jnp.int32),
            pltpu.VMEM((LCAP + 16,), jnp.int32),
            pltpu.VMEM((STRIP,), jnp.int32),
            pltpu.VMEM((STRIP,), jnp.int32),
            pltpu.VMEM((32,), jnp.int32),
        ] + [pltpu.VMEM((CO,), jnp.int32)] * 4 + [
            pltpu.VMEM((32, DH), jnp.float32),
            pltpu.VMEM_SHARED((NPAD, DH), jnp.float32),
            pltpu.SemaphoreType.DMA((4,)),
            pltpu.SemaphoreType.DMA((4,)),
            pltpu.SemaphoreType.DMA((2,)),
            pltpu.SemaphoreType.DMA,
            pltpu.SemaphoreType.DMA((2,)),
        ],
    )
    return f(edge_index[0], edge_index[1], *h4, *ew4)


def _post_body(x_ref, arl_ref, arh_ref, afl_ref, afh_ref, b_ref,
               ws_ref, wu_ref, ps_ref, pu_ref,
               w1_ref, b1_ref, w2_ref, b2_ref, w3_ref, b3_ref,
               feats_ref, gr_ref, sc_ref, sums, cnts):
    i = pl.program_id(0)

    @pl.when(i == 0)
    def _():
        sums[...] = jnp.zeros_like(sums)
        cnts[...] = jnp.zeros_like(cnts)

    xb = x_ref[...]
    wu = wu_ref[...]
    rep = jnp.maximum(
        _dot(arl_ref[0], wu[:DH, :]) + _dot(arh_ref[0], wu[DH:, :])
        + _dot(xb, ws_ref[...]), 0.0)
    pu = pu_ref[...]
    feats_ref[...] = jnp.maximum(
        _dot(afl_ref[0], pu[:DH, :]) + _dot(afh_ref[0], pu[DH:, :])
        + _dot(xb, ps_ref[...]), 0.0)

    bvals = b_ref[0, 0, :]
    onehot = (bvals[:, None]
              == jax.lax.broadcasted_iota(jnp.int32, (NB, G), 1)
              ).astype(jnp.float32)
    sums[...] += jax.lax.dot_general(
        onehot, rep, dimension_numbers=(((0,), (0,)), ((), ())),
        preferred_element_type=jnp.float32, precision=_HI)
    cnts[...] += jnp.broadcast_to(jnp.sum(onehot, axis=0)[:, None], (G, D))

    @pl.when(i == pl.num_programs(0) - 1)
    def _():
        gr = sums[...] / jnp.maximum(cnts[...], 1.0)
        gr_ref[...] = gr
        h1 = jnp.maximum(_dot(gr, w1_ref[...]) + b1_ref[...], 0.0)
        h2 = jnp.maximum(_dot(h1, w2_ref[...]) + b2_ref[...], 0.0)
        sc_ref[...] = _dot(h2, w3_ref[...])[:, :1] + b3_ref[...]


def _post(x, aggs, batch3, Ws, Wu, Ps, Pu, W1, b1, W2, b2, w3row, b3):
    n_blocks = N // NB
    full = lambda r, c: pl.BlockSpec((r, c), lambda i: (0, 0))
    return pl.pallas_call(
        _post_body,
        grid=(n_blocks,),
        in_specs=[
            pl.BlockSpec((NB, D), lambda i: (i, 0)),
            pl.BlockSpec((1, NB, DH), lambda i: (0, i, 0)),
            pl.BlockSpec((1, NB, DH), lambda i: (1, i, 0)),
            pl.BlockSpec((1, NB, DH), lambda i: (0, i, 0)),
            pl.BlockSpec((1, NB, DH), lambda i: (1, i, 0)),
            pl.BlockSpec((1, 1, NB), lambda i: (i, 0, 0)),
            full(D, D), full(D, D), full(D, D), full(D, D),
            full(D, D), full(1, D), full(D, D), full(1, D),
            full(D, DH), full(1, 1),
        ],
        out_specs=[
            pl.BlockSpec((NB, D), lambda i: (i, 0)),
            pl.BlockSpec((G, D), lambda i: (0, 0)),
            pl.BlockSpec((G, 1), lambda i: (0, 0)),
        ],
        out_shape=[
            jax.ShapeDtypeStruct((N, D), jnp.float32),
            jax.ShapeDtypeStruct((G, D), jnp.float32),
            jax.ShapeDtypeStruct((G, 1), jnp.float32),
        ],
        scratch_shapes=[
            pltpu.VMEM((G, D), jnp.float32),
            pltpu.VMEM((G, D), jnp.float32),
        ],
    )(x, aggs[0], aggs[0], aggs[1], aggs[1], batch3,
      Ws, Wu, Ps, Pu, W1, b1, W2, b2, w3row, b3)


def kernel(x, edge_index, edge_attr, batch,
           Wm, We, Ws, Wu, Pm, Pe, Ps, Pu, W1, b1, W2, b2, W3, b3):
    hr_st, hf_st, er_st, ef_st = _pre(x, edge_attr, Wm, Pm, We, Pe)
    aggs = _sc_edge(edge_index, (hr_st, hf_st), (er_st, ef_st))
    batch3 = batch.reshape(N // NB, 1, NB)
    w3pad = jnp.pad(W3, ((0, 0), (0, DH - W3.shape[1])))
    feats, gr, sc = _post(
        x, aggs, batch3, Ws, Wu, Ps, Pu,
        W1, b1.reshape(1, D), W2, b2.reshape(1, D),
        w3pad, b3.reshape(1, 1))
    return (sc[:, 0], gr, feats)


# speedup vs baseline: 2.4787x; 1.0452x over previous
import functools

import jax
import jax.numpy as jnp
from jax import lax
from jax.experimental import pallas as pl
from jax.experimental.pallas import tpu as pltpu
from jax.experimental.pallas import tpu_sc as plsc

N = 10000
E = 160000
D = 256
DH = 128
G = 64

NB = 400
EB = 2000
NSUB = 16
EPT = E // NSUB
CE = 40
NCH = EPT // CE
NPAD = 10240
RPT = NPAD // NSUB

_HI = jax.lax.Precision.HIGHEST


def _dot(a, b):
    return jnp.dot(a, b, preferred_element_type=jnp.float32)


def _pre_body(x_ref, ea_ref, wm_ref, pm_ref, we_ref, pe_ref,
              hr_ref, hf_ref, er_ref, ef_ref):
    i = pl.program_id(0)

    @pl.when(i < N // NB)
    def _():
        xb = x_ref[...]
        hr = _dot(xb, wm_ref[...])
        hr_ref[0] = hr[:, :DH]
        hr_ref[1] = hr[:, DH:]
        hf = _dot(xb, pm_ref[...])
        hf_ref[0] = hf[:, :DH]
        hf_ref[1] = hf[:, DH:]

    eb = ea_ref[...]
    er = _dot(eb, we_ref[...])
    er_ref[0] = er[:, :DH]
    er_ref[1] = er[:, DH:]
    ef = _dot(eb, pe_ref[...])
    ef_ref[0] = ef[:, :DH]
    ef_ref[1] = ef[:, DH:]


def _pre(x, edge_attr, Wm, Pm, We, Pe):
    de = edge_attr.shape[1]
    nh = N // NB
    full = lambda r, c: pl.BlockSpec((r, c), lambda i: (0, 0))
    hmap = lambda i: (jnp.minimum(i, nh - 1), 0)
    return pl.pallas_call(
        _pre_body,
        grid=(E // EB,),
        in_specs=[
            pl.BlockSpec((NB, D), hmap),
            pl.BlockSpec((EB, de), lambda i: (i, 0)),
            full(D, D), full(D, D), full(de, D), full(de, D),
        ],
        out_specs=[pl.BlockSpec((2, NB, DH), lambda i: (0, *hmap(i)))] * 2
        + [pl.BlockSpec((2, EB, DH), lambda i: (0, i, 0))] * 2,
        out_shape=[jax.ShapeDtypeStruct((2, N, DH), jnp.float32)] * 2
        + [jax.ShapeDtypeStruct((2, E, DH), jnp.float32)] * 2,
    )(x, edge_attr, Wm, Pm, We, Pe)


STRIP = 640
LCAP = 10624
CO = 32
NQUAD = LCAP // CO // 4


def _sc_body(src_hbm, dst_hbm,
             h_rep, h_ft, ew_rep, ew_ft,
             o_rep, o_ft,
             gatb, ewb, eidl, dstl, dbuf0, dbuf1, sbuf, s0, s1, s2, s3,
             zbuf, acc,
             src_sems, gat_sems, ew_sems, sc_sem, db_sems):
    srcv = [s0, s1, s2, s3]
    cid = lax.axis_index("c")
    sid = lax.axis_index("s")
    lo = sid * RPT

    def dstrip(sno):
        return dst_hbm.at[pl.ds(sno * STRIP, STRIP)]

    dbv = [dbuf0, dbuf1]

    def strip2(g, cnt):
        for b in range(2):
            sno = g * 2 + b
            pltpu.make_async_copy(dstrip(sno), dbv[b], db_sems.at[b]).wait()

            @pl.when(sno + 1 < E // STRIP)
            def _():
                pltpu.async_copy(dstrip(sno + 1), dbv[1 - b],
                                 db_sems.at[1 - b])

            cnt = scan_strip(sno, b, cnt)
        return cnt

    def scan_strip(sno, b, cnt):
        def vec(v, cnt):
            lane = lax.iota(jnp.int32, 16)
            d16 = dbv[b][pl.ds(v * 16, 16)]
            eid16 = lane + (sno * STRIP + v * 16)
            m = jnp.logical_and(d16 >= lo, d16 < lo + RPT)
            mi = m.astype(jnp.int32)
            cs = plsc.cumsum(mi)
            pos = jnp.where(m, cnt + cs - mi, LCAP + lane)
            plsc.store_scatter(eidl, [pos], eid16)
            plsc.store_scatter(dstl, [pos], d16)
            return cnt + cs[15]

        return lax.fori_loop(0, STRIP // 16, vec, cnt)

    pltpu.async_copy(dstrip(0), dbv[0], db_sems.at[0])
    cnt = lax.fori_loop(0, E // STRIP // 2, strip2, 0)
    eidl[pl.ds(cnt, 16)] = jnp.zeros((16,), jnp.int32)
    eidl[pl.ds(cnt + 16, 16)] = jnp.zeros((16,), jnp.int32)
    dstl[pl.ds(cnt, 16)] = jnp.full((16,), N, jnp.int32)
    dstl[pl.ds(cnt + 16, 16)] = jnp.full((16,), N, jnp.int32)
    nch = (cnt + CO - 1) // CO

    def process(h_hbm, ew_hbm, out_hbm):
        def start_src(c, b4):
            pltpu.async_copy(src_hbm.at[eidl.at[pl.ds(c * CO, CO)]],
                             srcv[b4], src_sems.at[b4])

        def wait_src(c, b4):
            pltpu.make_async_copy(src_hbm.at[eidl.at[pl.ds(c * CO, CO)]],
                                  srcv[b4], src_sems.at[b4]).wait()

        def start_gat(c, b4, b):
            pltpu.async_copy(h_hbm.at[cid].at[srcv[b4]], gatb.at[b4],
                             gat_sems.at[b4])
            pltpu.async_copy(ew_hbm.at[cid].at[eidl.at[pl.ds(c * CO, CO)]],
                             ewb.at[b], ew_sems.at[b])


        def wait_gat(c, b4, b):
            pltpu.make_async_copy(h_hbm.at[cid].at[srcv[b4]], gatb.at[b4],
                                  gat_sems.at[b4]).wait()
            pltpu.make_async_copy(ew_hbm.at[cid].at[eidl.at[pl.ds(c * CO, CO)]],
                                  ewb.at[b], ew_sems.at[b]).wait()


        def start_sc(c, b4):
            pltpu.async_copy(gatb.at[b4], acc.at[dstl.at[pl.ds(c * CO, CO)]],
                             sc_sem, add=True)

        def wait_sc_one():
            pltpu.make_async_copy(gatb.at[0], acc.at[dstl.at[pl.ds(0, CO)]],
                                  sc_sem).wait()

        def zrow(i, _):
            for j in range(DH // 16):
                zbuf[i, pl.ds(j * 16, 16)] = jnp.zeros((16,), jnp.float32)
            return 0

        lax.fori_loop(0, zbuf.shape[0], zrow, 0)
        for k in range(RPT // zbuf.shape[0]):
            pltpu.sync_copy(zbuf, acc.at[pl.ds(lo + k * zbuf.shape[0],
                                               zbuf.shape[0])])

        for c in range(4):
            @pl.when(c < nch)
            def _():
                start_src(c, c)
        for c in range(2):
            @pl.when(c < nch)
            def _():
                wait_src(c, c)
                start_gat(c, c, c % 2)

        def quad(g, _):
            for b in range(4):
                c = g * 4 + b

                @pl.when(c < nch)
                def _():
                    wait_gat(c, b, b % 2)

                    def row(i, _):
                        for j in range(DH // 16):
                            s = pl.ds(j * 16, 16)
                            gatb[b, i, s] = jnp.maximum(
                                gatb[b, i, s] + ewb[b % 2, i, s], 0.0)
                        return 0

                    lax.fori_loop(0, CO, row, 0)

                    @pl.when(c > 0)
                    def _():
                        wait_sc_one()

                    start_sc(c, b)

                    @pl.when(c + 2 < nch)
                    def _():
                        wait_src(c + 2, (b + 2) % 4)
                        start_gat(c + 2, (b + 2) % 4, b % 2)

                    @pl.when(c + 4 < nch)
                    def _():
                        start_src(c + 4, b)
            return 0

        lax.fori_loop(0, NQUAD, quad, 0)

        @pl.when(nch > 0)
        def _():
            wait_sc_one()

        pltpu.sync_copy(acc.at[pl.ds(lo, RPT)],
                        out_hbm.at[cid].at[pl.ds(lo, RPT)])

    process(h_rep, ew_rep, o_rep)
    process(h_ft, ew_ft, o_ft)


def _sc_edge(edge_index, h4, ew4):
    f = pl.kernel(
        _sc_body,
        out_type=[jax.ShapeDtypeStruct((2, NPAD, DH), jnp.float32)] * 2,
        mesh=plsc.VectorSubcoreMesh(core_axis_name="c", subcore_axis_name="s",
                                    num_cores=2, num_subcores=NSUB),
        compiler_params=pltpu.CompilerParams(needs_layout_passes=False),
        scratch_types=[
            pltpu.VMEM((4, CO, DH), jnp.float32),
            pltpu.VMEM((2, CO, DH), jnp.float32),
            pltpu.VMEM((LCAP + 16,), jnp.int32),
            pltpu.VMEM((LCAP + 16,), jnp.int32),
            pltpu.VMEM((STRIP,), jnp.int32),
            pltpu.VMEM((STRIP,), jnp.int32),
            pltpu.VMEM((32,), jnp.int32),
        ] + [pltpu.VMEM((CO,), jnp.int32)] * 4 + [
            pltpu.VMEM((8, DH), jnp.float32),
            pltpu.VMEM_SHARED((NPAD, DH), jnp.float32),
            pltpu.SemaphoreType.DMA((4,)),
            pltpu.SemaphoreType.DMA((4,)),
            pltpu.SemaphoreType.DMA((2,)),
            pltpu.SemaphoreType.DMA,
            pltpu.SemaphoreType.DMA((2,)),
        ],
    )
    return f(edge_index[0], edge_index[1], *h4, *ew4)


def _post_body(x_ref, arl_ref, arh_ref, afl_ref, afh_ref, b_ref,
               ws_ref, wu_ref, ps_ref, pu_ref,
               w1_ref, b1_ref, w2_ref, b2_ref, w3_ref, b3_ref,
               feats_ref, gr_ref, sc_ref, sums, cnts):
    i = pl.program_id(0)

    @pl.when(i == 0)
    def _():
        sums[...] = jnp.zeros_like(sums)
        cnts[...] = jnp.zeros_like(cnts)

    xb = x_ref[...]
    wu = wu_ref[...]
    rep = jnp.maximum(
        _dot(arl_ref[0], wu[:DH, :]) + _dot(arh_ref[0], wu[DH:, :])
        + _dot(xb, ws_ref[...]), 0.0)
    pu = pu_ref[...]
    feats_ref[...] = jnp.maximum(
        _dot(afl_ref[0], pu[:DH, :]) + _dot(afh_ref[0], pu[DH:, :])
        + _dot(xb, ps_ref[...]), 0.0)

    bvals = b_ref[0, 0, :]
    onehot = (bvals[:, None]
              == jax.lax.broadcasted_iota(jnp.int32, (NB, G), 1)
              ).astype(jnp.float32)
    sums[...] += jax.lax.dot_general(
        onehot, rep, dimension_numbers=(((0,), (0,)), ((), ())),
        preferred_element_type=jnp.float32, precision=_HI)
    cnts[...] += jnp.broadcast_to(jnp.sum(onehot, axis=0)[:, None], (G, D))

    @pl.when(i == pl.num_programs(0) - 1)
    def _():
        gr = sums[...] / jnp.maximum(cnts[...], 1.0)
        gr_ref[...] = gr
        h1 = jnp.maximum(_dot(gr, w1_ref[...]) + b1_ref[...], 0.0)
        h2 = jnp.maximum(_dot(h1, w2_ref[...]) + b2_ref[...], 0.0)
        sc_ref[...] = _dot(h2, w3_ref[...])[:, :1] + b3_ref[...]


def _post(x, aggs, batch3, Ws, Wu, Ps, Pu, W1, b1, W2, b2, w3row, b3):
    n_blocks = N // NB
    full = lambda r, c: pl.BlockSpec((r, c), lambda i: (0, 0))
    return pl.pallas_call(
        _post_body,
        grid=(n_blocks,),
        in_specs=[
            pl.BlockSpec((NB, D), lambda i: (i, 0)),
            pl.BlockSpec((1, NB, DH), lambda i: (0, i, 0)),
            pl.BlockSpec((1, NB, DH), lambda i: (1, i, 0)),
            pl.BlockSpec((1, NB, DH), lambda i: (0, i, 0)),
            pl.BlockSpec((1, NB, DH), lambda i: (1, i, 0)),
            pl.BlockSpec((1, 1, NB), lambda i: (i, 0, 0)),
            full(D, D), full(D, D), full(D, D), full(D, D),
            full(D, D), full(1, D), full(D, D), full(1, D),
            full(D, DH), full(1, 1),
        ],
        out_specs=[
            pl.BlockSpec((NB, D), lambda i: (i, 0)),
            pl.BlockSpec((G, D), lambda i: (0, 0)),
            pl.BlockSpec((G, 1), lambda i: (0, 0)),
        ],
        out_shape=[
            jax.ShapeDtypeStruct((N, D), jnp.float32),
            jax.ShapeDtypeStruct((G, D), jnp.float32),
            jax.ShapeDtypeStruct((G, 1), jnp.float32),
        ],
        scratch_shapes=[
            pltpu.VMEM((G, D), jnp.float32),
            pltpu.VMEM((G, D), jnp.float32),
        ],
    )(x, aggs[0], aggs[0], aggs[1], aggs[1], batch3,
      Ws, Wu, Ps, Pu, W1, b1, W2, b2, w3row, b3)


def kernel(x, edge_index, edge_attr, batch,
           Wm, We, Ws, Wu, Pm, Pe, Ps, Pu, W1, b1, W2, b2, W3, b3):
    hr_st, hf_st, er_st, ef_st = _pre(x, edge_attr, Wm, Pm, We, Pe)
    aggs = _sc_edge(edge_index, (hr_st, hf_st), (er_st, ef_st))
    batch3 = batch.reshape(N // NB, 1, NB)
    w3pad = jnp.pad(W3, ((0, 0), (0, DH - W3.shape[1])))
    feats, gr, sc = _post(
        x, aggs, batch3, Ws, Wu, Ps, Pu,
        W1, b1.reshape(1, D), W2, b2.reshape(1, D),
        w3pad, b3.reshape(1, 1))
    return (sc[:, 0], gr, feats)
